# Initial kernel scaffold; baseline (speedup 1.0000x reference)
import jax
import jax.numpy as jnp
from jax.experimental import pallas as pl


def kernel(x, edge_index, W1, b1, W2, b2):
    raise NotImplementedError("write your pallas kernel here")


# speedup vs baseline: 21.4397x; 21.4397x over previous
import functools
import jax
import jax.numpy as jnp
from jax import lax
from jax.experimental import pallas as pl
from jax.experimental.pallas import tpu as pltpu
from jax.experimental.pallas import tpu_sc as plsc

N = 10000
E = 320000
FIN = 128
HID = 128
CLS = 64
FH = 32
KPROP = 10
ALPHA = 0.1

NT = 16
B = 128
NCH = 160
EPT = NCH * B
EPAD = NT * EPT
NPAD = 10240
RT = NPAD // NT
DUMMY = N
RCH = 5


def _mlp_body(x_ref, w1_ref, b1_ref, w2_ref, b2_ref, out_ref):
    h = lax.dot_general(x_ref[...], w1_ref[...], (((1,), (1,)), ((), ())),
                        preferred_element_type=jnp.float32)
    h = jnp.maximum(h + b1_ref[...], 0.0)
    hh = lax.dot_general(h, w2_ref[...], (((1,), (1,)), ((), ())),
                         preferred_element_type=jnp.float32)
    hh = hh + b2_ref[...]
    out_ref[0] = hh[:, :FH]
    out_ref[1] = hh[:, FH:]


def _mlp(xp, W1, b1, W2, b2):
    return pl.pallas_call(
        _mlp_body,
        grid=(NPAD // RT,),
        in_specs=[
            pl.BlockSpec((RT, FIN), lambda i: (i, 0)),
            pl.BlockSpec((HID, FIN), lambda i: (0, 0)),
            pl.BlockSpec((1, HID), lambda i: (0, 0)),
            pl.BlockSpec((CLS, HID), lambda i: (0, 0)),
            pl.BlockSpec((1, CLS), lambda i: (0, 0)),
        ],
        out_specs=pl.BlockSpec((2, RT, FH), lambda i: (0, i, 0)),
        out_shape=jax.ShapeDtypeStruct((2, NPAD, FH), jnp.float32),
    )(xp, W1, b1, W2, b2)


def _rsqrt16(d):
    m = d
    r = jnp.full((16,), 1.0, jnp.float32)
    for _ in range(10):
        big = m >= 4.0
        m = jnp.where(big, m * 0.25, m)
        r = jnp.where(big, r * 0.5, r)
    y = jnp.full((16,), 0.70710678, jnp.float32)
    for _ in range(6):
        y = y * (1.5 - 0.5 * m * y * y)
    return y * r


def _sc_body(srcp, dstp, h0, out, g_s, agg_s,
             v_src, v_dst, v_dinv, v_gat, v_row, v_h0c, v_zero,
             sem):
    c = lax.axis_index("c")
    s = lax.axis_index("s")
    row0 = s * RT
    rows = pl.ds(row0, RT)

    pltpu.sync_copy(srcp.at[pl.ds(s * NCH, NCH)], v_src)
    pltpu.sync_copy(dstp.at[pl.ds(s * NCH, NCH)], v_dst)

    one16 = jnp.full((16,), 1.0, jnp.float32)
    zero16 = jnp.zeros((16,), jnp.float32)

    def fill_const(i, _):
        for half in range(2):
            sl = pl.ds(half * 16, 16)
            v_gat[i, sl] = one16
            v_zero[i, sl] = zero16
        return 0
    lax.fori_loop(0, B, fill_const, 0)

    for k in range(RCH):
        pltpu.sync_copy(v_gat, agg_s.at[pl.ds(row0 + k * B, B)])
    plsc.subcore_barrier()

    def deg_chunk(j, _):
        pltpu.sync_copy(v_gat, agg_s.at[v_dst.at[j]], add=True)
        return 0
    lax.fori_loop(0, NCH, deg_chunk, 0)
    plsc.subcore_barrier()

    for k in range(RCH):
        rslice = pl.ds(row0 + k * B, B)
        pltpu.sync_copy(agg_s.at[rslice], v_row)
        pltpu.sync_copy(h0.at[c, rslice], v_h0c)

        def dinv_row(i, _):
            dv = _rsqrt16(v_row[i, pl.ds(0, 16)])
            v_dinv[k * B + i, :] = dv
            for half in range(2):
                sl = pl.ds(half * 16, 16)
                v_row[i, sl] = dv * v_h0c[i, sl]
            return 0
        lax.fori_loop(0, B, dinv_row, 0)
        pltpu.sync_copy(v_row, g_s.at[rslice])
        pltpu.sync_copy(v_zero, agg_s.at[rslice])
    plsc.subcore_barrier()

    for _ in range(KPROP):
        def edge_chunk(j, _):
            pltpu.async_copy(g_s.at[v_src.at[j]], v_gat, sem).wait()
            pltpu.sync_copy(v_gat, agg_s.at[v_dst.at[j]], add=True)
            return 0
        lax.fori_loop(0, NCH, edge_chunk, 0)
        plsc.subcore_barrier()

        for k in range(RCH):
            rslice = pl.ds(row0 + k * B, B)
            pltpu.sync_copy(agg_s.at[rslice], v_gat)
            pltpu.sync_copy(g_s.at[rslice], v_row)
            pltpu.sync_copy(h0.at[c, rslice], v_h0c)

            def upd_row(i, _):
                dv = v_dinv[k * B + i, :]
                for half in range(2):
                    sl = pl.ds(half * 16, 16)
                    t = v_gat[i, sl] + v_row[i, sl]
                    hn = (1.0 - ALPHA) * dv * t + ALPHA * v_h0c[i, sl]
                    v_row[i, sl] = dv * hn
                return 0
            lax.fori_loop(0, B, upd_row, 0)
            pltpu.sync_copy(v_row, g_s.at[rslice])
            pltpu.sync_copy(v_zero, agg_s.at[rslice])
        plsc.subcore_barrier()

    for k in range(RCH):
        rslice = pl.ds(row0 + k * B, B)
        pltpu.sync_copy(g_s.at[rslice], v_row)

        def final_row(i, _):
            dv = v_dinv[k * B + i, :]
            for half in range(2):
                sl = pl.ds(half * 16, 16)
                v_row[i, sl] = v_row[i, sl] / dv
            return 0
        lax.fori_loop(0, B, final_row, 0)
        pltpu.sync_copy(v_row, out.at[c, rslice])


def _propagate(srcp, dstp, h0):
    mesh = plsc.VectorSubcoreMesh(core_axis_name="c", subcore_axis_name="s")
    return pl.kernel(
        _sc_body,
        out_type=jax.ShapeDtypeStruct((2, NPAD, FH), jnp.float32),
        mesh=mesh,
        scratch_types=[
            pltpu.VMEM_SHARED((NPAD, FH), jnp.float32),
            pltpu.VMEM_SHARED((NPAD, FH), jnp.float32),
            pltpu.VMEM((NCH, B), jnp.int32),
            pltpu.VMEM((NCH, B), jnp.int32),
            pltpu.VMEM((RT, 16), jnp.float32),
            pltpu.VMEM((B, FH), jnp.float32),
            pltpu.VMEM((B, FH), jnp.float32),
            pltpu.VMEM((B, FH), jnp.float32),
            pltpu.VMEM((B, FH), jnp.float32),
            pltpu.SemaphoreType.DMA,
        ],
        compiler_params=pltpu.CompilerParams(use_tc_tiling_on_sc=False),
    )(srcp, dstp, h0)


def _lsm_body(a_ref, out_ref):
    h = jnp.concatenate([a_ref[0], a_ref[1]], axis=1)
    m = jnp.max(h, axis=1, keepdims=True)
    e = jnp.exp(h - m)
    lse = jnp.log(jnp.sum(e, axis=1, keepdims=True)) + m
    out_ref[...] = h - lse


def _log_softmax(hk):
    rb = 400
    return pl.pallas_call(
        _lsm_body,
        grid=(N // rb,),
        in_specs=[pl.BlockSpec((2, rb, FH), lambda i: (0, i, 0))],
        out_specs=pl.BlockSpec((rb, CLS), lambda i: (i, 0)),
        out_shape=jax.ShapeDtypeStruct((N, CLS), jnp.float32),
    )(hk)


@jax.jit
def kernel(x, edge_index, W1, b1, W2, b2):
    xp = jnp.pad(x, ((0, NPAD - N), (0, 0)))
    pad = jnp.full((EPAD - E,), DUMMY, jnp.int32)
    srcp = jnp.concatenate([edge_index[0], pad]).reshape(NT * NCH, B)
    dstp = jnp.concatenate([edge_index[1], pad]).reshape(NT * NCH, B)
    h0 = _mlp(xp, W1, b1.reshape(1, HID), W2, b2.reshape(1, CLS))
    hk = _propagate(srcp, dstp, h0)
    return _log_softmax(hk)


# speedup vs baseline: 26.6564x; 1.2433x over previous
import functools
import jax
import jax.numpy as jnp
from jax import lax
from jax.experimental import pallas as pl
from jax.experimental.pallas import tpu as pltpu
from jax.experimental.pallas import tpu_sc as plsc

N = 10000
E = 320000
FIN = 128
HID = 128
CLS = 64
FH = 32
KPROP = 10
ALPHA = 0.1

NT = 16
B = 128
NCH = 160
EPT = NCH * B
EPAD = NT * EPT
NPAD = 10240
RT = NPAD // NT
DUMMY = N
RCH = 5


def _mlp_body(x_ref, w1_ref, b1_ref, w2_ref, b2_ref, out_ref):
    h = lax.dot_general(x_ref[...], w1_ref[...], (((1,), (1,)), ((), ())),
                        preferred_element_type=jnp.float32)
    h = jnp.maximum(h + b1_ref[...], 0.0)
    hh = lax.dot_general(h, w2_ref[...], (((1,), (1,)), ((), ())),
                         preferred_element_type=jnp.float32)
    hh = hh + b2_ref[...]
    out_ref[0] = hh[:, :FH]
    out_ref[1] = hh[:, FH:]


def _mlp(xp, W1, b1, W2, b2):
    return pl.pallas_call(
        _mlp_body,
        grid=(NPAD // RT,),
        in_specs=[
            pl.BlockSpec((RT, FIN), lambda i: (i, 0)),
            pl.BlockSpec((HID, FIN), lambda i: (0, 0)),
            pl.BlockSpec((1, HID), lambda i: (0, 0)),
            pl.BlockSpec((CLS, HID), lambda i: (0, 0)),
            pl.BlockSpec((1, CLS), lambda i: (0, 0)),
        ],
        out_specs=pl.BlockSpec((2, RT, FH), lambda i: (0, i, 0)),
        out_shape=jax.ShapeDtypeStruct((2, NPAD, FH), jnp.float32),
    )(xp, W1, b1, W2, b2)


def _rsqrt16(d):
    m = d
    r = jnp.full((16,), 1.0, jnp.float32)
    for _ in range(10):
        big = m >= 4.0
        m = jnp.where(big, m * 0.25, m)
        r = jnp.where(big, r * 0.5, r)
    y = jnp.full((16,), 0.70710678, jnp.float32)
    for _ in range(6):
        y = y * (1.5 - 0.5 * m * y * y)
    return y * r


def _sc_body(srcp, dstp, h0, out, g_s, agg_s,
             v_src, v_dst, v_dinv, v_gat, v_gat2, v_row, v_h0c, v_zero,
             sem):
    c = lax.axis_index("c")
    s = lax.axis_index("s")
    row0 = s * RT
    rows = pl.ds(row0, RT)

    pltpu.sync_copy(srcp.at[pl.ds(s * NCH, NCH)], v_src)
    pltpu.sync_copy(dstp.at[pl.ds(s * NCH, NCH)], v_dst)

    one16 = jnp.full((16,), 1.0, jnp.float32)
    zero16 = jnp.zeros((16,), jnp.float32)

    def fill_const(i, _):
        for half in range(2):
            sl = pl.ds(half * 16, 16)
            v_gat[i, sl] = one16
            v_zero[i, sl] = zero16
        return 0
    lax.fori_loop(0, B, fill_const, 0)

    for k in range(RCH):
        pltpu.sync_copy(v_gat, agg_s.at[pl.ds(row0 + k * B, B)])
    plsc.subcore_barrier()

    def deg_chunk(j, _):
        pltpu.sync_copy(v_gat, agg_s.at[v_dst.at[j]], add=True)
        return 0
    lax.fori_loop(0, NCH, deg_chunk, 0)
    plsc.subcore_barrier()

    for k in range(RCH):
        rslice = pl.ds(row0 + k * B, B)
        pltpu.sync_copy(agg_s.at[rslice], v_row)
        pltpu.sync_copy(h0.at[c, rslice], v_h0c)

        def dinv_row(i, _):
            dv = _rsqrt16(v_row[i, pl.ds(0, 16)])
            v_dinv[k * B + i, :] = dv
            for half in range(2):
                sl = pl.ds(half * 16, 16)
                v_row[i, sl] = dv * v_h0c[i, sl]
            return 0
        lax.fori_loop(0, B, dinv_row, 0)
        pltpu.sync_copy(v_row, g_s.at[rslice])
        pltpu.sync_copy(v_zero, agg_s.at[rslice])
    plsc.subcore_barrier()

    def _gat(j, buf):
        return pltpu.make_async_copy(g_s.at[v_src.at[j]], buf, sem)

    NQ = NCH // 2
    for _ in range(KPROP):
        _gat(0, v_gat).start()

        def edge_pair(q, _):
            jA = 2 * q
            _gat(jA, v_gat).wait()
            _gat(jA + 1, v_gat2).start()
            pltpu.sync_copy(v_gat, agg_s.at[v_dst.at[jA]], add=True)
            _gat(jA + 1, v_gat2).wait()

            @pl.when(q < NQ - 1)
            def _():
                _gat(jA + 2, v_gat).start()
            pltpu.sync_copy(v_gat2, agg_s.at[v_dst.at[jA + 1]], add=True)
            return 0
        lax.fori_loop(0, NQ, edge_pair, 0)
        plsc.subcore_barrier()

        for k in range(RCH):
            rslice = pl.ds(row0 + k * B, B)
            pltpu.sync_copy(agg_s.at[rslice], v_gat)
            pltpu.sync_copy(g_s.at[rslice], v_row)
            pltpu.sync_copy(h0.at[c, rslice], v_h0c)

            def upd_row(i, _):
                dv = v_dinv[k * B + i, :]
                for half in range(2):
                    sl = pl.ds(half * 16, 16)
                    t = v_gat[i, sl] + v_row[i, sl]
                    hn = (1.0 - ALPHA) * dv * t + ALPHA * v_h0c[i, sl]
                    v_row[i, sl] = dv * hn
                return 0
            lax.fori_loop(0, B, upd_row, 0)
            pltpu.sync_copy(v_row, g_s.at[rslice])
            pltpu.sync_copy(v_zero, agg_s.at[rslice])
        plsc.subcore_barrier()

    for k in range(RCH):
        rslice = pl.ds(row0 + k * B, B)
        pltpu.sync_copy(g_s.at[rslice], v_row)

        def final_row(i, _):
            dv = v_dinv[k * B + i, :]
            for half in range(2):
                sl = pl.ds(half * 16, 16)
                v_row[i, sl] = v_row[i, sl] / dv
            return 0
        lax.fori_loop(0, B, final_row, 0)
        pltpu.sync_copy(v_row, out.at[c, rslice])


def _propagate(srcp, dstp, h0):
    mesh = plsc.VectorSubcoreMesh(core_axis_name="c", subcore_axis_name="s")
    return pl.kernel(
        _sc_body,
        out_type=jax.ShapeDtypeStruct((2, NPAD, FH), jnp.float32),
        mesh=mesh,
        scratch_types=[
            pltpu.VMEM_SHARED((NPAD, FH), jnp.float32),
            pltpu.VMEM_SHARED((NPAD, FH), jnp.float32),
            pltpu.VMEM((NCH, B), jnp.int32),
            pltpu.VMEM((NCH, B), jnp.int32),
            pltpu.VMEM((RT, 16), jnp.float32),
            pltpu.VMEM((B, FH), jnp.float32),
            pltpu.VMEM((B, FH), jnp.float32),
            pltpu.VMEM((B, FH), jnp.float32),
            pltpu.VMEM((B, FH), jnp.float32),
            pltpu.VMEM((B, FH), jnp.float32),
            pltpu.SemaphoreType.DMA,
        ],
        compiler_params=pltpu.CompilerParams(use_tc_tiling_on_sc=False),
    )(srcp, dstp, h0)


def _lsm_body(a_ref, out_ref):
    h = jnp.concatenate([a_ref[0], a_ref[1]], axis=1)
    m = jnp.max(h, axis=1, keepdims=True)
    e = jnp.exp(h - m)
    lse = jnp.log(jnp.sum(e, axis=1, keepdims=True)) + m
    out_ref[...] = h - lse


def _log_softmax(hk):
    rb = 400
    return pl.pallas_call(
        _lsm_body,
        grid=(N // rb,),
        in_specs=[pl.BlockSpec((2, rb, FH), lambda i: (0, i, 0))],
        out_specs=pl.BlockSpec((rb, CLS), lambda i: (i, 0)),
        out_shape=jax.ShapeDtypeStruct((N, CLS), jnp.float32),
    )(hk)


@jax.jit
def kernel(x, edge_index, W1, b1, W2, b2):
    xp = jnp.pad(x, ((0, NPAD - N), (0, 0)))
    pad = jnp.full((EPAD - E,), DUMMY, jnp.int32)
    srcp = jnp.concatenate([edge_index[0], pad]).reshape(NT * NCH, B)
    dstp = jnp.concatenate([edge_index[1], pad]).reshape(NT * NCH, B)
    h0 = _mlp(xp, W1, b1.reshape(1, HID), W2, b2.reshape(1, CLS))
    hk = _propagate(srcp, dstp, h0)
    return _log_softmax(hk)
